# baseline (speedup 1.0000x reference)
import functools

import jax
import jax.numpy as jnp
from jax import lax
from jax.experimental import pallas as pl
from jax.experimental.pallas import tpu as pltpu
from jax.experimental.pallas import tpu_sc as plsc

N_CLS = 91
N_WORKERS = 32
ROWS_PER_B = 1000
ROWS_PER_W = 500
PAD_SLOTS = 2048
BLK = 16
RPAD = 512
TBL_HALF = 16 * N_CLS * 4 * ROWS_PER_B


def _make_worker(half):
    def _worker(tci_hbm, tb_hbm, table_hbm, out_hbm,
                tci_v, idx_v, rows_v, tb_v, part_v, sem):
        wid = lax.axis_index("c") * 16 + lax.axis_index("s")
        b = half * 16 + (wid >> 1)
        roi0 = (wid & 1) * ROWS_PER_W

        pltpu.sync_copy(tci_hbm.at[pl.ds(b * ROWS_PER_B, ROWS_PER_B)], tci_v)
        for k in range(4):
            pltpu.sync_copy(
                tb_hbm.at[pl.ds((b * 4 + k) * ROWS_PER_B, ROWS_PER_B)],
                tb_v.at[pl.ds(k * 1024, ROWS_PER_B)],
            )

        iota = lax.iota(jnp.int32, 16)
        base = (wid >> 1) * (N_CLS * 4 * ROWS_PER_B)

        def flat_of(r_b, k):
            t = plsc.load_gather(tci_v, [r_b])
            cls = jnp.clip(t, 0, N_CLS - 1)
            return base + cls * (4 * ROWS_PER_B) + k * ROWS_PER_B + r_b, t

        def idx_body(j, carry):
            s = j * 16 + iota
            r_b = roi0 + jnp.minimum(s >> 2, ROWS_PER_W - 1)
            f, _ = flat_of(r_b, s & 3)
            idx_v[pl.ds(pl.multiple_of(j * 16, 16), 16)] = f >> 4
            return carry

        lax.fori_loop(0, PAD_SLOTS // 16, idx_body, 0)

        pltpu.async_copy(table_hbm.at[idx_v], rows_v, sem).wait()

        zero = jnp.zeros((16,), jnp.float32)

        def body(j, carry):
            a, n = carry
            e = j * 16 + iota
            k = e >> 9
            r = e & (RPAD - 1)
            valid = r < ROWS_PER_W
            rc = jnp.minimum(r, ROWS_PER_W - 1)
            f, t = flat_of(roi0 + rc, k)
            m = jnp.logical_and(t > 0, valid)
            sel = plsc.load_gather(rows_v, [rc * 4 + k, f & (BLK - 1)])
            tb = plsc.load_gather(tb_v, [k * 1024 + roi0 + rc])
            d = jnp.abs(tb - sel)
            l = jnp.where(d < 1.0, 0.5 * d * d, d - 0.5)
            a = a + jnp.where(m, l, 0.0)
            n = n + jnp.where(m, 1.0, 0.0)
            return a, n

        acc, cnt = lax.fori_loop(0, (4 * RPAD) // 16, body, (zero, zero))

        part_v[0, :] = acc
        part_v[1, :] = cnt
        pltpu.sync_copy(part_v, out_hbm.at[wid])

    return _worker


@jax.jit
def _sc_loss(tci, tb, table_a, table_b):
    mesh = plsc.VectorSubcoreMesh(core_axis_name="c", subcore_axis_name="s")
    outs = []
    for half in range(2):
        run = functools.partial(
            pl.kernel,
            mesh=mesh,
            compiler_params=pltpu.CompilerParams(
                needs_layout_passes=False, use_tc_tiling_on_sc=False
            ),
            out_type=jax.ShapeDtypeStruct((N_WORKERS, 2, 16), jnp.float32),
            scratch_types=[
                pltpu.VMEM((ROWS_PER_B,), jnp.int32),
                pltpu.VMEM((PAD_SLOTS,), jnp.int32),
                pltpu.VMEM((PAD_SLOTS, BLK), jnp.float32),
                pltpu.VMEM((4 * 1024,), jnp.float32),
                pltpu.VMEM((2, 16), jnp.float32),
                pltpu.SemaphoreType.DMA,
            ],
            name=f"sc_loss_h{half}",
        )(_make_worker(half))
        outs.append(run(tci, tb, table_a if half == 0 else table_b))
    return outs


def kernel(target_bbox, target_class_ids, pred_bbox):
    tci = target_class_ids.reshape(-1).astype(jnp.int32)
    tb = jnp.transpose(target_bbox, (0, 2, 1)).reshape(-1)
    halves = [
        jnp.transpose(pred_bbox[h * 16:(h + 1) * 16], (0, 2, 3, 1)).reshape(-1, BLK)
        for h in range(2)
    ]
    pa, pb = _sc_loss(tci, tb, *halves)
    parts = pa + pb
    total = parts[:, 0, :].sum()
    count = parts[:, 1, :].sum()
    return total / count


# speedup vs baseline: 1.2804x; 1.2804x over previous
import functools

import jax
import jax.numpy as jnp
from jax import lax
from jax.experimental import pallas as pl
from jax.experimental.pallas import tpu as pltpu
from jax.experimental.pallas import tpu_sc as plsc

N_CLS = 91
N_WORKERS = 32
ROWS_PER_W = 1000
N_ELEM = ROWS_PER_W * 4
PAD_SLOTS = 4096
BLK = 16
RPAD = 1024


def _worker(tci_hbm, tb_hbm, table_hbm, out_hbm,
            tci_v, idx_v, rows_v, tb_v, part_v, sem):
    wid = lax.axis_index("c") * 16 + lax.axis_index("s")
    row0 = wid * ROWS_PER_W

    pltpu.sync_copy(tci_hbm.at[pl.ds(row0, ROWS_PER_W)], tci_v)
    for k in range(4):
        pltpu.sync_copy(
            tb_hbm.at[pl.ds((wid * 4 + k) * ROWS_PER_W, ROWS_PER_W)],
            tb_v.at[pl.ds(k * RPAD, ROWS_PER_W)],
        )

    iota = lax.iota(jnp.int32, 16)
    base = wid * (N_CLS * 4 * ROWS_PER_W)

    def flat_of(r, k):
        t = plsc.load_gather(tci_v, [r])
        cls = jnp.clip(t, 0, N_CLS - 1)
        return base + cls * (4 * ROWS_PER_W) + k * ROWS_PER_W + r, t

    def idx_body(j, carry):
        s = j * 16 + iota
        f, _ = flat_of(jnp.minimum(s >> 2, ROWS_PER_W - 1), s & 3)
        idx_v[pl.ds(pl.multiple_of(j * 16, 16), 16)] = f >> 4
        return carry

    lax.fori_loop(0, PAD_SLOTS // 16, idx_body, 0)

    pltpu.async_copy(table_hbm.at[idx_v], rows_v, sem).wait()

    zero = jnp.zeros((16,), jnp.float32)

    def body(j, carry):
        a, n = carry
        e = j * 16 + iota
        k = e >> 10
        r = e & (RPAD - 1)
        valid = r < ROWS_PER_W
        rc = jnp.minimum(r, ROWS_PER_W - 1)
        f, t = flat_of(rc, k)
        m = jnp.logical_and(t > 0, valid)
        sel = plsc.load_gather(rows_v, [rc * 4 + k, f & (BLK - 1)])
        tb = tb_v[pl.ds(pl.multiple_of(j * 16, 16), 16)]
        d = jnp.abs(tb - sel)
        l = jnp.where(d < 1.0, 0.5 * d * d, d - 0.5)
        a = a + jnp.where(m, l, 0.0)
        n = n + jnp.where(m, 1.0, 0.0)
        return a, n

    acc, cnt = lax.fori_loop(0, (4 * RPAD) // 16, body, (zero, zero))

    part_v[0, :] = acc
    part_v[1, :] = cnt
    pltpu.sync_copy(part_v, out_hbm.at[wid])


@jax.jit
def _sc_loss(tci, tb, table):
    mesh = plsc.VectorSubcoreMesh(core_axis_name="c", subcore_axis_name="s")
    run = functools.partial(
        pl.kernel,
        mesh=mesh,
        compiler_params=pltpu.CompilerParams(
            needs_layout_passes=False, use_tc_tiling_on_sc=False
        ),
        out_type=jax.ShapeDtypeStruct((N_WORKERS, 2, 16), jnp.float32),
        scratch_types=[
            pltpu.VMEM((ROWS_PER_W,), jnp.int32),
            pltpu.VMEM((PAD_SLOTS,), jnp.int32),
            pltpu.VMEM((PAD_SLOTS, BLK), jnp.float32),
            pltpu.VMEM((4 * RPAD,), jnp.float32),
            pltpu.VMEM((2, 16), jnp.float32),
            pltpu.SemaphoreType.DMA,
        ],
    )(_worker)
    return run(tci, tb, table)


def kernel(target_bbox, target_class_ids, pred_bbox):
    tci = target_class_ids.reshape(-1).astype(jnp.int32)
    tb = jnp.transpose(target_bbox, (0, 2, 1)).reshape(-1)
    table = jnp.transpose(pred_bbox, (0, 2, 3, 1)).reshape(-1, BLK)
    parts = _sc_loss(tci, tb, table)
    total = parts[:, 0, :].sum()
    count = parts[:, 1, :].sum()
    return total / count


# speedup vs baseline: 1.2805x; 1.0000x over previous
import functools

import jax
import jax.numpy as jnp
from jax import lax
from jax.experimental import pallas as pl
from jax.experimental.pallas import tpu as pltpu
from jax.experimental.pallas import tpu_sc as plsc

N_CLS = 91
N_WORKERS = 32
ROWS_PER_W = 1000
N_ELEM = ROWS_PER_W * 4
PAD_SLOTS = 4096
BLK = 16
RPAD = 1024


def _worker(tci_hbm, tb_hbm, table_hbm, out_hbm,
            tci_v, idx_v, f_v, rows_v, tb_v, part_v, sem):
    wid = lax.axis_index("c") * 16 + lax.axis_index("s")
    row0 = wid * ROWS_PER_W

    pltpu.sync_copy(tci_hbm.at[pl.ds(row0, ROWS_PER_W)], tci_v)
    for k in range(4):
        pltpu.sync_copy(
            tb_hbm.at[pl.ds((wid * 4 + k) * ROWS_PER_W, ROWS_PER_W)],
            tb_v.at[pl.ds(k * RPAD, ROWS_PER_W)],
        )

    iota = lax.iota(jnp.int32, 16)
    base = wid * (N_CLS * 4 * ROWS_PER_W)

    def flat_of(r, k):
        t = plsc.load_gather(tci_v, [r])
        cls = jnp.clip(t, 0, N_CLS - 1)
        return base + cls * (4 * ROWS_PER_W) + k * ROWS_PER_W + r, t

    def idx_body(j, carry):
        s = j * 16 + iota
        r = s >> 2
        rc = jnp.minimum(r, ROWS_PER_W - 1)
        f, t = flat_of(rc, s & 3)
        idx_v[pl.ds(pl.multiple_of(j * 16, 16), 16)] = f >> 4
        live = jnp.logical_and(t > 0, r < ROWS_PER_W)
        f_v[pl.ds(pl.multiple_of(j * 16, 16), 16)] = jnp.where(live, f, -1)
        return carry

    lax.fori_loop(0, PAD_SLOTS // 16, idx_body, 0)

    pltpu.async_copy(table_hbm.at[idx_v], rows_v, sem).wait()

    zero = jnp.zeros((16,), jnp.float32)

    def body(j, carry):
        a, n = carry
        s = j * 16 + iota
        f = f_v[pl.ds(pl.multiple_of(j * 16, 16), 16)]
        m = f >= 0
        sel = plsc.load_gather(rows_v, [s, f & (BLK - 1)])
        tb = plsc.load_gather(tb_v, [(s & 3) * RPAD + (s >> 2)])
        d = jnp.abs(tb - sel)
        l = jnp.where(d < 1.0, 0.5 * d * d, d - 0.5)
        a = a + jnp.where(m, l, 0.0)
        n = n + jnp.where(m, 1.0, 0.0)
        return a, n

    acc, cnt = lax.fori_loop(0, PAD_SLOTS // 16, body, (zero, zero))

    part_v[0, :] = acc
    part_v[1, :] = cnt
    pltpu.sync_copy(part_v, out_hbm.at[wid])


@jax.jit
def _sc_loss(tci, tb, table):
    mesh = plsc.VectorSubcoreMesh(core_axis_name="c", subcore_axis_name="s")
    run = functools.partial(
        pl.kernel,
        mesh=mesh,
        compiler_params=pltpu.CompilerParams(
            needs_layout_passes=False, use_tc_tiling_on_sc=False
        ),
        out_type=jax.ShapeDtypeStruct((N_WORKERS, 2, 16), jnp.float32),
        scratch_types=[
            pltpu.VMEM((ROWS_PER_W,), jnp.int32),
            pltpu.VMEM((PAD_SLOTS,), jnp.int32),
            pltpu.VMEM((PAD_SLOTS,), jnp.int32),
            pltpu.VMEM((PAD_SLOTS, BLK), jnp.float32),
            pltpu.VMEM((4 * RPAD,), jnp.float32),
            pltpu.VMEM((2, 16), jnp.float32),
            pltpu.SemaphoreType.DMA,
        ],
    )(_worker)
    return run(tci, tb, table)


def kernel(target_bbox, target_class_ids, pred_bbox):
    tci = target_class_ids.reshape(-1).astype(jnp.int32)
    tb = jnp.transpose(target_bbox, (0, 2, 1)).reshape(-1)
    table = jnp.transpose(pred_bbox, (0, 2, 3, 1)).reshape(-1, BLK)
    parts = _sc_loss(tci, tb, table)
    total = parts[:, 0, :].sum()
    count = parts[:, 1, :].sum()
    return total / count
